# Initial kernel scaffold; baseline (speedup 1.0000x reference)
import jax
import jax.numpy as jnp
from jax.experimental import pallas as pl


def kernel(tokens, table):
    raise NotImplementedError("write your pallas kernel here")


# speedup vs baseline: 1.2555x; 1.2555x over previous
import functools
import math

import jax
import jax.numpy as jnp
from jax import lax
from jax.experimental import pallas as pl
from jax.experimental.pallas import tpu as pltpu
from jax.experimental.pallas import tpu_sc as plsc

_EMB = 32
_SCALE = math.sqrt(float(_EMB))
_NC = 2
_NS = 16
_NW = _NC * _NS
_CHUNK = 128


def _gather_kernel(n_chunks, table_hbm, idx_hbm, out_hbm, idx_v, rows_v, sem):
    wid = lax.axis_index("s") * _NC + lax.axis_index("c")
    pltpu.sync_copy(idx_hbm.at[pl.ds(wid * n_chunks, n_chunks)], idx_v)

    def chunk_body(j, carry):
        buf = rows_v.at[0]
        pltpu.async_copy(table_hbm.at[idx_v.at[j]], buf, sem).wait()

        def scale_body(i, c):
            v0 = buf[i, pl.ds(0, 16)]
            v1 = buf[i, pl.ds(16, 16)]
            buf[i, pl.ds(0, 16)] = v0 * _SCALE
            buf[i, pl.ds(16, 16)] = v1 * _SCALE
            return c

        lax.fori_loop(0, _CHUNK, scale_body, 0, unroll=4)
        row_base = (wid * n_chunks + j) * _CHUNK
        pltpu.sync_copy(buf, out_hbm.at[pl.ds(row_base, _CHUNK)])
        return carry

    lax.fori_loop(0, n_chunks, chunk_body, 0)


@functools.partial(jax.jit, static_argnums=(2,))
def _embed(idx, table, n_chunks):
    mesh = plsc.VectorSubcoreMesh(
        core_axis_name="c", subcore_axis_name="s",
        num_cores=_NC, num_subcores=_NS)
    n_rows = idx.shape[0] * idx.shape[1]
    run = pl.kernel(
        functools.partial(_gather_kernel, n_chunks),
        out_type=jax.ShapeDtypeStruct((n_rows, _EMB), jnp.float32),
        mesh=mesh,
        scratch_types=[
            pltpu.VMEM((n_chunks, _CHUNK), jnp.int32),
            pltpu.VMEM((1, _CHUNK, _EMB), jnp.float32),
            pltpu.SemaphoreType.DMA,
        ],
        compiler_params=pltpu.CompilerParams(use_tc_tiling_on_sc=False),
    )
    return run(table, idx)


def kernel(tokens, table):
    b = tokens.size
    assert b % (_NW * _CHUNK) == 0
    n_chunks = b // (_NW * _CHUNK)
    idx = tokens.reshape(_NW * n_chunks, _CHUNK).astype(jnp.int32)
    out = _embed(idx, table, n_chunks)
    return out.reshape(*tokens.shape, _EMB)


# speedup vs baseline: 1.2664x; 1.0087x over previous
import functools
import math

import jax
import jax.numpy as jnp
from jax import lax
from jax.experimental import pallas as pl
from jax.experimental.pallas import tpu as pltpu
from jax.experimental.pallas import tpu_sc as plsc

_EMB = 32
_SCALE = math.sqrt(float(_EMB))
_NC = 2
_NS = 16
_NW = _NC * _NS
_CHUNK = 128
_NBUF = 4


def _gather_kernel(n_chunks, table_hbm, idx_hbm, out_hbm,
                   idx_v, gbuf, sbuf, gsem, ssem):
    wid = lax.axis_index("s") * _NC + lax.axis_index("c")
    pltpu.sync_copy(idx_hbm.at[pl.ds(wid * n_chunks, n_chunks)], idx_v)
    row0 = wid * n_chunks * _CHUNK

    def start_gather(j, b):
        pltpu.async_copy(table_hbm.at[idx_v.at[j]], gbuf.at[b], gsem.at[b])

    def wait_gather(j, b):
        pltpu.make_async_copy(
            table_hbm.at[idx_v.at[j]], gbuf.at[b], gsem.at[b]).wait()

    def scale(b):
        def body(i, c):
            sbuf[b, i, pl.ds(0, 16)] = gbuf[b, i, pl.ds(0, 16)] * _SCALE
            sbuf[b, i, pl.ds(16, 16)] = gbuf[b, i, pl.ds(16, 16)] * _SCALE
            return c
        lax.fori_loop(0, _CHUNK, body, 0, unroll=8)

    def start_store(j, b):
        pltpu.async_copy(
            sbuf.at[b], out_hbm.at[pl.ds(row0 + j * _CHUNK, _CHUNK)],
            ssem.at[b])

    def wait_store(j, b):
        pltpu.make_async_copy(
            sbuf.at[b], out_hbm.at[pl.ds(row0 + j * _CHUNK, _CHUNK)],
            ssem.at[b]).wait()

    n_outer = n_chunks // _NBUF
    for b in range(_NBUF):
        start_gather(b, b)
    for b in range(_NBUF):
        wait_gather(b, b)
        scale(b)
        start_store(b, b)
        start_gather(_NBUF + b, b)
    def outer(g, c):
        for b in range(_NBUF):
            j = g * _NBUF + b
            wait_gather(j, b)
            wait_store(j - _NBUF, b)
            scale(b)
            start_store(j, b)
            start_gather(j + _NBUF, b)
        return c
    lax.fori_loop(1, n_outer - 1, outer, 0)
    for b in range(_NBUF):
        j = (n_outer - 1) * _NBUF + b
        wait_gather(j, b)
        wait_store(j - _NBUF, b)
        scale(b)
        start_store(j, b)
    for b in range(_NBUF):
        wait_store((n_outer - 1) * _NBUF + b, b)


@functools.partial(jax.jit, static_argnums=(2,))
def _embed(idx, table, n_chunks):
    mesh = plsc.VectorSubcoreMesh(
        core_axis_name="c", subcore_axis_name="s",
        num_cores=_NC, num_subcores=_NS)
    n_rows = idx.shape[0] * idx.shape[1]
    run = pl.kernel(
        functools.partial(_gather_kernel, n_chunks),
        out_type=jax.ShapeDtypeStruct((n_rows, _EMB), jnp.float32),
        mesh=mesh,
        scratch_types=[
            pltpu.VMEM((n_chunks, _CHUNK), jnp.int32),
            pltpu.VMEM((_NBUF, _CHUNK, _EMB), jnp.float32),
            pltpu.VMEM((_NBUF, _CHUNK, _EMB), jnp.float32),
            pltpu.SemaphoreType.DMA((_NBUF,)),
            pltpu.SemaphoreType.DMA((_NBUF,)),
        ],
        compiler_params=pltpu.CompilerParams(use_tc_tiling_on_sc=False),
    )
    return run(table, idx)


def kernel(tokens, table):
    b = tokens.size
    assert b % (_NW * _CHUNK * _NBUF) == 0
    n_chunks = b // (_NW * _CHUNK)
    idx = tokens.reshape(_NW * n_chunks, _CHUNK).astype(jnp.int32)
    out = _embed(idx, table, n_chunks)
    return out.reshape(*tokens.shape, _EMB)


# speedup vs baseline: 1.4243x; 1.1247x over previous
import functools
import math

import jax
import jax.numpy as jnp
from jax import lax
from jax.experimental import pallas as pl
from jax.experimental.pallas import tpu as pltpu
from jax.experimental.pallas import tpu_sc as plsc

_EMB = 32
_SCALE = math.sqrt(float(_EMB))
_NC = 2
_NS = 16
_NW = _NC * _NS
_CHUNK = 128
_NBUF = 8


def _gather_kernel(n_chunks, table_hbm, idx_hbm, out_hbm,
                   idx_v, gbuf, sbuf, gsem, ssem):
    wid = lax.axis_index("s") * _NC + lax.axis_index("c")
    pltpu.sync_copy(idx_hbm.at[pl.ds(wid * n_chunks, n_chunks)], idx_v)
    row0 = wid * n_chunks * _CHUNK

    def start_gather(j, b):
        pltpu.async_copy(table_hbm.at[idx_v.at[j]], gbuf.at[b], gsem.at[b])

    def wait_gather(j, b):
        pltpu.make_async_copy(
            table_hbm.at[idx_v.at[j]], gbuf.at[b], gsem.at[b]).wait()

    def scale(b):
        sbuf[b] = gbuf[b] * _SCALE

    def start_store(j, b):
        pltpu.async_copy(
            sbuf.at[b], out_hbm.at[pl.ds(row0 + j * _CHUNK, _CHUNK)],
            ssem.at[b])

    def wait_store(j, b):
        pltpu.make_async_copy(
            sbuf.at[b], out_hbm.at[pl.ds(row0 + j * _CHUNK, _CHUNK)],
            ssem.at[b]).wait()

    n_outer = n_chunks // _NBUF
    for b in range(_NBUF):
        start_gather(b, b)
    for b in range(_NBUF):
        wait_gather(b, b)
        scale(b)
        start_store(b, b)
        start_gather(_NBUF + b, b)
    def outer(g, c):
        for b in range(_NBUF):
            j = g * _NBUF + b
            wait_gather(j, b)
            wait_store(j - _NBUF, b)
            scale(b)
            start_store(j, b)
            start_gather(j + _NBUF, b)
        return c
    lax.fori_loop(1, n_outer - 1, outer, 0)
    for b in range(_NBUF):
        j = (n_outer - 1) * _NBUF + b
        wait_gather(j, b)
        wait_store(j - _NBUF, b)
        scale(b)
        start_store(j, b)
    for b in range(_NBUF):
        wait_store((n_outer - 1) * _NBUF + b, b)


@functools.partial(jax.jit, static_argnums=(2,))
def _embed(idx, table, n_chunks):
    mesh = plsc.VectorSubcoreMesh(
        core_axis_name="c", subcore_axis_name="s",
        num_cores=_NC, num_subcores=_NS)
    n_rows = idx.shape[0] * idx.shape[1]
    run = pl.kernel(
        functools.partial(_gather_kernel, n_chunks),
        out_type=jax.ShapeDtypeStruct((n_rows, _EMB), jnp.float32),
        mesh=mesh,
        scratch_types=[
            pltpu.VMEM((n_chunks, _CHUNK), jnp.int32),
            pltpu.VMEM((_NBUF, _CHUNK, _EMB), jnp.float32),
            pltpu.VMEM((_NBUF, _CHUNK, _EMB), jnp.float32),
            pltpu.SemaphoreType.DMA((_NBUF,)),
            pltpu.SemaphoreType.DMA((_NBUF,)),
        ],
        compiler_params=pltpu.CompilerParams(use_tc_tiling_on_sc=False),
    )
    return run(table, idx)


def kernel(tokens, table):
    b = tokens.size
    assert b % (_NW * _CHUNK * _NBUF) == 0
    n_chunks = b // (_NW * _CHUNK)
    idx = tokens.reshape(_NW * n_chunks, _CHUNK).astype(jnp.int32)
    out = _embed(idx, table, n_chunks)
    return out.reshape(*tokens.shape, _EMB)
